# baseline (speedup 1.0000x reference)
import functools

import jax
import jax.numpy as jnp
from jax import lax
from jax.experimental import pallas as pl
from jax.experimental.pallas import tpu as pltpu
from jax.experimental.pallas import tpu_sc as plsc

STEPS = 4096
CHANNELS = 512
NQ = 65536

NC = 2
NS = 16
L = 16
NW = NC * NS
QPW = NQ // NW
C = 16
NCHUNK = QPW // C
D = 4

_mesh = plsc.VectorSubcoreMesh(
    core_axis_name="c", subcore_axis_name="s", num_cores=NC, num_subcores=NS
)


@functools.partial(
    pl.kernel,
    out_type=jax.ShapeDtypeStruct((NQ, CHANNELS), jnp.float32),
    mesh=_mesh,
    scratch_types=[
        pltpu.VMEM((QPW,), jnp.float32),
        pltpu.VMEM((D, C), jnp.int32),
        pltpu.VMEM((D, C), jnp.int32),
        pltpu.VMEM((D, C), jnp.float32),
        pltpu.VMEM((D, C, CHANNELS), jnp.float32),
        pltpu.VMEM((D, C, CHANNELS), jnp.float32),
        pltpu.VMEM((2, C, CHANNELS), jnp.float32),
        pltpu.SemaphoreType.DMA,
        pltpu.SemaphoreType.DMA,
        pltpu.SemaphoreType.DMA,
        pltpu.SemaphoreType.DMA,
        pltpu.SemaphoreType.DMA,
        pltpu.SemaphoreType.DMA,
        pltpu.SemaphoreType.DMA,
        pltpu.SemaphoreType.DMA,
        pltpu.SemaphoreType.DMA,
        pltpu.SemaphoreType.DMA,
    ],
)
def _interp_kernel(t_hbm, control_hbm, out_hbm,
                   t_v, idx0, idx1, w_v, rows0, rows1, outb,
                   sg00, sg01, sg02, sg03, sg10, sg11, sg12, sg13, soa, sob):
    wid = lax.axis_index("s") * NC + lax.axis_index("c")
    base = wid * QPW
    pltpu.sync_copy(t_hbm.at[pl.ds(base, QPW)], t_v)
    sg0 = (sg00, sg01, sg02, sg03)
    sg1 = (sg10, sg11, sg12, sg13)
    so = (soa, sob)

    def issue(g, b):
        qb = g * C
        for k in range(C // L):
            tv = t_v[pl.ds(qb + k * L, L)]
            xs = tv * jnp.float32(STEPS - 1)
            ji = xs.astype(jnp.int32)
            ji = jnp.maximum(jnp.minimum(ji, STEPS - 2), 0)
            idx0[b, pl.ds(k * L, L)] = ji
            idx1[b, pl.ds(k * L, L)] = ji + 1
            w_v[b, pl.ds(k * L, L)] = xs - ji.astype(jnp.float32)
        pltpu.async_copy(control_hbm.at[idx0.at[b]], rows0.at[b], sg0[b])
        pltpu.async_copy(control_hbm.at[idx1.at[b]], rows1.at[b], sg1[b])

    def blend(b, ob):
        for qg in range(C // L):
            wvec = w_v[b, pl.ds(qg * L, L)]
            for l in range(L):
                wspl = jnp.full((L,), wvec[l], jnp.float32)
                q = qg * L + l

                @plsc.parallel_loop(0, CHANNELS, step=L, unroll=16)
                def ch_body(c):
                    r0 = rows0[b, q, pl.ds(c, L)]
                    r1 = rows1[b, q, pl.ds(c, L)]
                    outb[ob, q, pl.ds(c, L)] = r0 + wspl * (r1 - r0)

    for g in range(D - 1):
        issue(g, g)

    def outer(gg, carry):
        for u in range(D):
            g = gg * D + u
            b = u
            ob = u % 2
            pltpu.make_async_copy(
                control_hbm.at[idx0.at[b]], rows0.at[b], sg0[b]).wait()
            pltpu.make_async_copy(
                control_hbm.at[idx1.at[b]], rows1.at[b], sg1[b]).wait()

            @pl.when(g + D - 1 < NCHUNK)
            def _():
                issue(g + D - 1, (u + D - 1) % D)

            @pl.when(g > 1)
            def _():
                pltpu.make_async_copy(
                    outb.at[ob], out_hbm.at[pl.ds(base, C)], so[ob]).wait()

            blend(b, ob)
            pltpu.async_copy(
                outb.at[ob], out_hbm.at[pl.ds(base + g * C, C)], so[ob])
        return carry

    lax.fori_loop(0, NCHUNK // D, outer, 0)
    pltpu.make_async_copy(outb.at[0], out_hbm.at[pl.ds(base, C)], soa).wait()
    pltpu.make_async_copy(outb.at[1], out_hbm.at[pl.ds(base, C)], sob).wait()


def kernel(t, control):
    return _interp_kernel(t, control)


# speedup vs baseline: 1.0801x; 1.0801x over previous
import functools

import jax
import jax.numpy as jnp
from jax import lax
from jax.experimental import pallas as pl
from jax.experimental.pallas import tpu as pltpu
from jax.experimental.pallas import tpu_sc as plsc

STEPS = 4096
CHANNELS = 512
NQ = 65536

NC = 2
NS = 16
L = 16
NW = NC * NS
QPW = NQ // NW
C = 16
NCHUNK = QPW // C
D = 4

_mesh = plsc.VectorSubcoreMesh(
    core_axis_name="c", subcore_axis_name="s", num_cores=NC, num_subcores=NS
)


@functools.partial(
    pl.kernel,
    out_type=jax.ShapeDtypeStruct((NQ, CHANNELS), jnp.float32),
    mesh=_mesh,
    scratch_types=[
        pltpu.VMEM((QPW,), jnp.float32),
        pltpu.VMEM((D, C), jnp.int32),
        pltpu.VMEM((D, C), jnp.int32),
        pltpu.VMEM((D, C), jnp.float32),
        pltpu.VMEM((D, C, CHANNELS), jnp.float32),
        pltpu.VMEM((D, C, CHANNELS), jnp.float32),
        pltpu.VMEM((2, C, CHANNELS), jnp.float32),
        pltpu.SemaphoreType.DMA,
        pltpu.SemaphoreType.DMA,
        pltpu.SemaphoreType.DMA,
        pltpu.SemaphoreType.DMA,
        pltpu.SemaphoreType.DMA,
        pltpu.SemaphoreType.DMA,
        pltpu.SemaphoreType.DMA,
        pltpu.SemaphoreType.DMA,
        pltpu.SemaphoreType.DMA,
        pltpu.SemaphoreType.DMA,
    ],
)
def _interp_kernel(t_hbm, control_hbm, out_hbm,
                   t_v, idx0, idx1, w_v, rows0, rows1, outb,
                   sg00, sg01, sg02, sg03, sg10, sg11, sg12, sg13, soa, sob):
    wid = lax.axis_index("s") * NC + lax.axis_index("c")
    base = wid * QPW
    pltpu.sync_copy(t_hbm.at[pl.ds(base, QPW)], t_v)
    sg0 = (sg00, sg01, sg02, sg03)
    sg1 = (sg10, sg11, sg12, sg13)
    so = (soa, sob)

    def issue(g, b):
        qb = g * C
        for k in range(C // L):
            tv = t_v[pl.ds(qb + k * L, L)]
            xs = tv * jnp.float32(STEPS - 1)
            ji = xs.astype(jnp.int32)
            ji = jnp.maximum(jnp.minimum(ji, STEPS - 2), 0)
            idx0[b, pl.ds(k * L, L)] = ji
            idx1[b, pl.ds(k * L, L)] = ji + 1
            w_v[b, pl.ds(k * L, L)] = xs - ji.astype(jnp.float32)
        pltpu.async_copy(control_hbm.at[idx0.at[b]], rows0.at[b], sg0[b])
        pltpu.async_copy(control_hbm.at[idx1.at[b]], rows1.at[b], sg1[b])

    def blend(b, ob):
        for qg in range(C // L):
            wvec = w_v[b, pl.ds(qg * L, L)]
            for l in range(L):
                wspl = jnp.full((L,), wvec[l], jnp.float32)
                q = qg * L + l

                @plsc.parallel_loop(0, CHANNELS, step=L, unroll=8)
                def ch_body(c):
                    r0 = rows0[b, q, pl.ds(c, L)]
                    r1 = rows1[b, q, pl.ds(c, L)]
                    outb[ob, q, pl.ds(c, L)] = r0 + wspl * (r1 - r0)

    for g in range(D - 1):
        issue(g, g)

    def outer(gg, carry):
        for u in range(D):
            g = gg * D + u
            b = u
            ob = u % 2
            pltpu.make_async_copy(
                control_hbm.at[idx0.at[b]], rows0.at[b], sg0[b]).wait()
            pltpu.make_async_copy(
                control_hbm.at[idx1.at[b]], rows1.at[b], sg1[b]).wait()

            @pl.when(g + D - 1 < NCHUNK)
            def _():
                issue(g + D - 1, (u + D - 1) % D)

            @pl.when(g > 1)
            def _():
                pltpu.make_async_copy(
                    outb.at[ob], out_hbm.at[pl.ds(base, C)], so[ob]).wait()

            blend(b, ob)
            pltpu.async_copy(
                outb.at[ob], out_hbm.at[pl.ds(base + g * C, C)], so[ob])
        return carry

    lax.fori_loop(0, NCHUNK // D, outer, 0)
    pltpu.make_async_copy(outb.at[0], out_hbm.at[pl.ds(base, C)], soa).wait()
    pltpu.make_async_copy(outb.at[1], out_hbm.at[pl.ds(base, C)], sob).wait()


def kernel(t, control):
    return _interp_kernel(t, control)


# speedup vs baseline: 1.1069x; 1.0248x over previous
import functools

import jax
import jax.numpy as jnp
from jax import lax
from jax.experimental import pallas as pl
from jax.experimental.pallas import tpu as pltpu
from jax.experimental.pallas import tpu_sc as plsc

STEPS = 4096
CHANNELS = 512
NQ = 65536

NC = 2
NS = 16
L = 16
NW = NC * NS
QPW = NQ // NW
C = 16
NCHUNK = QPW // C
D = 4

_mesh = plsc.VectorSubcoreMesh(
    core_axis_name="c", subcore_axis_name="s", num_cores=NC, num_subcores=NS
)


@functools.partial(
    pl.kernel,
    out_type=jax.ShapeDtypeStruct((NQ, CHANNELS), jnp.float32),
    mesh=_mesh,
    scratch_types=[
        pltpu.VMEM((QPW,), jnp.float32),
        pltpu.VMEM((D, C), jnp.int32),
        pltpu.VMEM((D, C), jnp.int32),
        pltpu.VMEM((D, C), jnp.float32),
        pltpu.VMEM((D, C, CHANNELS), jnp.float32),
        pltpu.VMEM((D, C, CHANNELS), jnp.float32),
        pltpu.VMEM((2, C, CHANNELS), jnp.float32),
        pltpu.SemaphoreType.DMA,
        pltpu.SemaphoreType.DMA,
        pltpu.SemaphoreType.DMA,
        pltpu.SemaphoreType.DMA,
        pltpu.SemaphoreType.DMA,
        pltpu.SemaphoreType.DMA,
        pltpu.SemaphoreType.DMA,
        pltpu.SemaphoreType.DMA,
        pltpu.SemaphoreType.DMA,
        pltpu.SemaphoreType.DMA,
    ],
)
def _interp_kernel(t_hbm, control_hbm, out_hbm,
                   t_v, idx0, idx1, w_v, rows0, rows1, outb,
                   sg00, sg01, sg02, sg03, sg10, sg11, sg12, sg13, soa, sob):
    wid = lax.axis_index("s") * NC + lax.axis_index("c")
    base = wid * QPW
    pltpu.sync_copy(t_hbm.at[pl.ds(base, QPW)], t_v)
    sg0 = (sg00, sg01, sg02, sg03)
    sg1 = (sg10, sg11, sg12, sg13)
    so = (soa, sob)

    def issue(g, b):
        qb = g * C
        for k in range(C // L):
            tv = t_v[pl.ds(qb + k * L, L)]
            xs = tv * jnp.float32(STEPS - 1)
            ji = xs.astype(jnp.int32)
            ji = jnp.maximum(jnp.minimum(ji, STEPS - 2), 0)
            idx0[b, pl.ds(k * L, L)] = ji
            idx1[b, pl.ds(k * L, L)] = ji + 1
            w_v[b, pl.ds(k * L, L)] = xs - ji.astype(jnp.float32)
        pltpu.async_copy(control_hbm.at[idx0.at[b]], rows0.at[b], sg0[b])
        pltpu.async_copy(control_hbm.at[idx1.at[b]], rows1.at[b], sg1[b])

    def blend(b, ob):
        for qg in range(C // L):
            wvec = w_v[b, pl.ds(qg * L, L)]
            for l in range(L):
                wspl = jnp.full((L,), wvec[l], jnp.float32)
                q = qg * L + l

                @plsc.parallel_loop(0, CHANNELS, step=L, unroll=4)
                def ch_body(c):
                    r0 = rows0[b, q, pl.ds(c, L)]
                    r1 = rows1[b, q, pl.ds(c, L)]
                    outb[ob, q, pl.ds(c, L)] = r0 + wspl * (r1 - r0)

    for g in range(D - 1):
        issue(g, g)

    def outer(gg, carry):
        for u in range(D):
            g = gg * D + u
            b = u
            ob = u % 2
            pltpu.make_async_copy(
                control_hbm.at[idx0.at[b]], rows0.at[b], sg0[b]).wait()
            pltpu.make_async_copy(
                control_hbm.at[idx1.at[b]], rows1.at[b], sg1[b]).wait()

            @pl.when(g + D - 1 < NCHUNK)
            def _():
                issue(g + D - 1, (u + D - 1) % D)

            @pl.when(g > 1)
            def _():
                pltpu.make_async_copy(
                    outb.at[ob], out_hbm.at[pl.ds(base, C)], so[ob]).wait()

            blend(b, ob)
            pltpu.async_copy(
                outb.at[ob], out_hbm.at[pl.ds(base + g * C, C)], so[ob])
        return carry

    lax.fori_loop(0, NCHUNK // D, outer, 0)
    pltpu.make_async_copy(outb.at[0], out_hbm.at[pl.ds(base, C)], soa).wait()
    pltpu.make_async_copy(outb.at[1], out_hbm.at[pl.ds(base, C)], sob).wait()


def kernel(t, control):
    return _interp_kernel(t, control)
